# baseline (device time: 8458 ns/iter reference)
import jax
import jax.numpy as jnp
from jax import lax
from jax.experimental import pallas as pl
from jax.experimental.pallas import tpu as pltpu

NCHUNK = 2


def kernel(x, dy, gamma):
    m_per, d = x.shape
    cm = m_per // NCHUNK

    def body(x_hbm, dy_hbm, gamma_hbm, out_ref,
             xv_ref, dyv_ref, acc_ref, comm_ref,
             in_sems, send_sem, recv_sem):
        my_x = lax.axis_index("x")
        my_y = lax.axis_index("y")

        barrier_sem = pltpu.get_barrier_semaphore()
        pl.semaphore_signal(
            barrier_sem, inc=1,
            device_id=(my_x, 1 - my_y),
            device_id_type=pl.DeviceIdType.MESH,
        )

        for c in range(NCHUNK):
            sl = pl.ds(c * cm, cm)
            pltpu.make_async_copy(x_hbm.at[sl, :], xv_ref.at[sl, :],
                                  in_sems.at[2 * c]).start()
            pltpu.make_async_copy(dy_hbm.at[sl, :], dyv_ref.at[sl, :],
                                  in_sems.at[2 * c + 1]).start()

        for c in range(NCHUNK):
            sl = pl.ds(c * cm, cm)
            pltpu.make_async_copy(x_hbm.at[sl, :], xv_ref.at[sl, :],
                                  in_sems.at[2 * c]).wait()
            pltpu.make_async_copy(dy_hbm.at[sl, :], dyv_ref.at[sl, :],
                                  in_sems.at[2 * c + 1]).wait()
            xv = xv_ref[sl, :]
            dyv = dyv_ref[sl, :]
            mu = jnp.mean(xv, axis=1, keepdims=True)
            diff = xv - mu
            var = jnp.mean(diff * diff, axis=1, keepdims=True)
            rstd = lax.rsqrt(var + 1e-5)
            dgamma = jnp.sum(dyv * (diff * rstd), axis=0, keepdims=True)
            dbeta = jnp.sum(dyv, axis=0, keepdims=True)
            partial = jnp.concatenate([dgamma, dbeta], axis=0)
            if c == 0:
                acc_ref[:, :] = partial
            else:
                acc_ref[:, :] = acc_ref[:, :] + partial

        pl.semaphore_wait(barrier_sem, 1)
        rdma = pltpu.make_async_remote_copy(
            src_ref=acc_ref,
            dst_ref=comm_ref,
            send_sem=send_sem,
            recv_sem=recv_sem,
            device_id=(my_x, 1 - my_y),
            device_id_type=pl.DeviceIdType.MESH,
        )
        rdma.start()
        rdma.wait()
        out_ref[:, :] = acc_ref[:, :] + comm_ref[:, :]

    x = pltpu.with_memory_space_constraint(x, pltpu.MemorySpace.HBM)
    dy = pltpu.with_memory_space_constraint(dy, pltpu.MemorySpace.HBM)
    gamma = pltpu.with_memory_space_constraint(gamma, pltpu.MemorySpace.HBM)

    return pl.pallas_call(
        body,
        out_shape=jax.ShapeDtypeStruct((2, d), jnp.float32),
        in_specs=[
            pl.BlockSpec(memory_space=pltpu.MemorySpace.HBM),
            pl.BlockSpec(memory_space=pltpu.MemorySpace.HBM),
            pl.BlockSpec(memory_space=pltpu.MemorySpace.HBM),
        ],
        out_specs=pl.BlockSpec(memory_space=pltpu.VMEM),
        scratch_shapes=[
            pltpu.VMEM((m_per, d), jnp.float32),
            pltpu.VMEM((m_per, d), jnp.float32),
            pltpu.VMEM((2, d), jnp.float32),
            pltpu.VMEM((2, d), jnp.float32),
            pltpu.SemaphoreType.DMA((2 * NCHUNK,)),
            pltpu.SemaphoreType.DMA,
            pltpu.SemaphoreType.DMA,
        ],
        compiler_params=pltpu.CompilerParams(collective_id=0),
    )(x, dy, gamma)


# device time: 8037 ns/iter; 1.0524x vs baseline; 1.0524x over previous
import jax
import jax.numpy as jnp
from jax import lax
from jax.experimental import pallas as pl
from jax.experimental.pallas import tpu as pltpu

NCHUNK = 4


def kernel(x, dy, gamma):
    m_per, d = x.shape
    hm = m_per // 2
    cm = hm // NCHUNK

    def body(x_hbm, dy_hbm, gamma_hbm, out_ref,
             xv_ref, dyv_ref, acc_ref, comm_ref,
             in_sems, send_sems, recv_sems):
        my_x = lax.axis_index("x")
        my_y = lax.axis_index("y")
        partners = [
            (my_x, 1 - my_y),
            (1 - my_x, my_y),
            (1 - my_x, 1 - my_y),
        ]

        barrier_sem = pltpu.get_barrier_semaphore()
        for p in partners:
            pl.semaphore_signal(
                barrier_sem, inc=1,
                device_id=p, device_id_type=pl.DeviceIdType.MESH,
            )

        row0 = my_x * hm
        for c in range(NCHUNK):
            src = pl.ds(row0 + c * cm, cm)
            dst = pl.ds(c * cm, cm)
            pltpu.make_async_copy(x_hbm.at[src, :], xv_ref.at[dst, :],
                                  in_sems.at[2 * c]).start()
            pltpu.make_async_copy(dy_hbm.at[src, :], dyv_ref.at[dst, :],
                                  in_sems.at[2 * c + 1]).start()

        for c in range(NCHUNK):
            src = pl.ds(row0 + c * cm, cm)
            dst = pl.ds(c * cm, cm)
            pltpu.make_async_copy(x_hbm.at[src, :], xv_ref.at[dst, :],
                                  in_sems.at[2 * c]).wait()
            pltpu.make_async_copy(dy_hbm.at[src, :], dyv_ref.at[dst, :],
                                  in_sems.at[2 * c + 1]).wait()
            xv = xv_ref[dst, :]
            dyv = dyv_ref[dst, :]
            mu = jnp.mean(xv, axis=1, keepdims=True)
            diff = xv - mu
            var = jnp.mean(diff * diff, axis=1, keepdims=True)
            rstd = lax.rsqrt(var + 1e-5)
            dgamma = jnp.sum(dyv * (diff * rstd), axis=0, keepdims=True)
            dbeta = jnp.sum(dyv, axis=0, keepdims=True)
            partial = jnp.concatenate([dgamma, dbeta], axis=0)
            if c == 0:
                acc_ref[:, :] = partial
            else:
                acc_ref[:, :] = acc_ref[:, :] + partial

        pl.semaphore_wait(barrier_sem, 3)
        rdmas = []
        for k, p in enumerate(partners):
            rdma = pltpu.make_async_remote_copy(
                src_ref=acc_ref,
                dst_ref=comm_ref.at[k],
                send_sem=send_sems.at[k],
                recv_sem=recv_sems.at[k],
                device_id=p,
                device_id_type=pl.DeviceIdType.MESH,
            )
            rdma.start()
            rdmas.append(rdma)
        for rdma in rdmas:
            rdma.wait()

        out_ref[:, :] = (
            (acc_ref[:, :] + comm_ref[0, :, :])
            + (comm_ref[1, :, :] + comm_ref[2, :, :])
        )

    x = pltpu.with_memory_space_constraint(x, pltpu.MemorySpace.HBM)
    dy = pltpu.with_memory_space_constraint(dy, pltpu.MemorySpace.HBM)
    gamma = pltpu.with_memory_space_constraint(gamma, pltpu.MemorySpace.HBM)

    return pl.pallas_call(
        body,
        out_shape=jax.ShapeDtypeStruct((2, d), jnp.float32),
        in_specs=[
            pl.BlockSpec(memory_space=pltpu.MemorySpace.HBM),
            pl.BlockSpec(memory_space=pltpu.MemorySpace.HBM),
            pl.BlockSpec(memory_space=pltpu.MemorySpace.HBM),
        ],
        out_specs=pl.BlockSpec(memory_space=pltpu.VMEM),
        scratch_shapes=[
            pltpu.VMEM((hm, d), jnp.float32),
            pltpu.VMEM((hm, d), jnp.float32),
            pltpu.VMEM((2, d), jnp.float32),
            pltpu.VMEM((3, 2, d), jnp.float32),
            pltpu.SemaphoreType.DMA((2 * NCHUNK,)),
            pltpu.SemaphoreType.DMA((3,)),
            pltpu.SemaphoreType.DMA((3,)),
        ],
        compiler_params=pltpu.CompilerParams(collective_id=0),
    )(x, dy, gamma)
